# initial kernel scaffold (unmeasured)
import functools

import jax
import jax.numpy as jnp
from jax import lax
from jax.experimental import pallas as pl
from jax.experimental.pallas import tpu as pltpu

N_DEV = 8
M = 2048
D = 2048
HB = 256
CHUNK = M // N_DEV


def _compute_body(x_ref, wg_ref, wu_ref, wd_ref, out_ref):
    k = pl.program_id(0)
    xb = x_ref[...]
    g = jnp.dot(xb, wg_ref[...].astype(jnp.bfloat16),
                preferred_element_type=jnp.float32)
    u = jnp.dot(xb, wu_ref[...].astype(jnp.bfloat16),
                preferred_element_type=jnp.float32)
    h = (g * (u * (1.0 / (1.0 + jnp.exp(-u))))).astype(jnp.bfloat16)
    p = jnp.dot(h, wd_ref[...].astype(jnp.bfloat16),
                preferred_element_type=jnp.float32)

    @pl.when(k == 0)
    def _():
        out_ref[...] = p

    @pl.when(k != 0)
    def _():
        out_ref[...] += p


def _allreduce_body(p_ref, out_ref, send_buf, recv_buf, send_sems, recv_sems,
                    credit_sem):
    my = lax.axis_index("i")
    left = jnp.mod(my - 1, N_DEV)
    right = jnp.mod(my + 1, N_DEV)

    barrier = pltpu.get_barrier_semaphore()
    for nbr in (left, right):
        pl.semaphore_signal(barrier, inc=1, device_id=(nbr,),
                            device_id_type=pl.DeviceIdType.MESH)
    pl.semaphore_wait(barrier, 2)

    out_ref[...] = p_ref[...]

    for t in range(2 * (N_DEV - 1)):
        slot = t % 2
        if t < N_DEV - 1:
            s = t
            send_idx = jnp.mod(my - s, N_DEV)
            recv_idx = jnp.mod(my - s - 1, N_DEV)
        else:
            s = t - (N_DEV - 1)
            send_idx = jnp.mod(my + 1 - s, N_DEV)
            recv_idx = jnp.mod(my - s, N_DEV)

        send_buf[slot] = out_ref[pl.ds(send_idx * CHUNK, CHUNK), :].astype(
            jnp.bfloat16)
        if t >= 1:
            pl.semaphore_wait(credit_sem, 1)
        rdma = pltpu.make_async_remote_copy(
            src_ref=send_buf.at[slot],
            dst_ref=recv_buf.at[slot],
            send_sem=send_sems.at[slot],
            recv_sem=recv_sems.at[slot],
            device_id=(right,),
            device_id_type=pl.DeviceIdType.MESH,
        )
        rdma.start()
        rdma.wait()

        if t < N_DEV - 1:
            out_ref[pl.ds(recv_idx * CHUNK, CHUNK), :] += (
                recv_buf[slot].astype(jnp.float32))
        else:
            out_ref[pl.ds(recv_idx * CHUNK, CHUNK), :] = (
                recv_buf[slot].astype(jnp.float32))
        if t < 2 * (N_DEV - 1) - 1:
            pl.semaphore_signal(credit_sem, inc=1, device_id=(left,),
                                device_id_type=pl.DeviceIdType.MESH)


def kernel(x, Wg, Wu, Wd):
    m, k = x.shape
    h_per = Wg.shape[1]
    d = Wd.shape[1]
    n_steps = h_per // HB

    x_bf = x.astype(jnp.bfloat16)

    partial = pl.pallas_call(
        _compute_body,
        grid=(n_steps,),
        in_specs=[
            pl.BlockSpec((m, k), lambda i: (0, 0)),
            pl.BlockSpec((k, HB), lambda i: (0, i)),
            pl.BlockSpec((k, HB), lambda i: (0, i)),
            pl.BlockSpec((HB, d), lambda i: (i, 0)),
        ],
        out_specs=pl.BlockSpec((m, d), lambda i: (0, 0)),
        out_shape=jax.ShapeDtypeStruct((m, d), jnp.float32),
        compiler_params=pltpu.CompilerParams(
            dimension_semantics=("arbitrary",)),
    )(x_bf, Wg, Wu, Wd)

    return pl.pallas_call(
        _allreduce_body,
        out_shape=jax.ShapeDtypeStruct((m, d), jnp.float32),
        in_specs=[pl.BlockSpec(memory_space=pltpu.VMEM)],
        out_specs=pl.BlockSpec(memory_space=pltpu.VMEM),
        scratch_shapes=[
            pltpu.VMEM((2, CHUNK, d), jnp.bfloat16),
            pltpu.VMEM((2, CHUNK, d), jnp.bfloat16),
            pltpu.SemaphoreType.DMA((2,)),
            pltpu.SemaphoreType.DMA((2,)),
            pltpu.SemaphoreType.REGULAR,
        ],
        compiler_params=pltpu.CompilerParams(collective_id=0),
    )(partial)


# baseline (device time: 394634 ns/iter reference)
import functools

import jax
import jax.numpy as jnp
from jax import lax
from jax.experimental import pallas as pl
from jax.experimental.pallas import tpu as pltpu

N_DEV = 8
M = 2048
D = 2048
HB = 256
CHUNK = M // N_DEV


def _compute_body(x_ref, wg_ref, wu_ref, wd_ref, out_ref):
    k = pl.program_id(0)
    xb = x_ref[...]
    g = jnp.dot(xb, wg_ref[...].astype(jnp.bfloat16),
                preferred_element_type=jnp.float32)
    u = jnp.dot(xb, wu_ref[...].astype(jnp.bfloat16),
                preferred_element_type=jnp.float32)
    h = (g * (u * (1.0 / (1.0 + jnp.exp(-u))))).astype(jnp.bfloat16)
    p = jnp.dot(h, wd_ref[...].astype(jnp.bfloat16),
                preferred_element_type=jnp.float32)

    @pl.when(k == 0)
    def _():
        out_ref[...] = p

    @pl.when(k != 0)
    def _():
        out_ref[...] += p


def _allreduce_body(p_ref, out_ref, send_buf, recv_buf, send_sems, recv_sems,
                    credit_sem):
    my = lax.axis_index("i")
    left = jnp.mod(my - 1, N_DEV)
    right = jnp.mod(my + 1, N_DEV)

    barrier = pltpu.get_barrier_semaphore()
    for nbr in (left, right):
        pl.semaphore_signal(barrier, inc=1, device_id=(nbr,),
                            device_id_type=pl.DeviceIdType.MESH)
    pl.semaphore_wait(barrier, 2)

    out_ref[...] = p_ref[...]

    for t in range(2 * (N_DEV - 1)):
        slot = t % 2
        if t < N_DEV - 1:
            s = t
            send_idx = jnp.mod(my - s, N_DEV)
            recv_idx = jnp.mod(my - s - 1, N_DEV)
        else:
            s = t - (N_DEV - 1)
            send_idx = jnp.mod(my + 1 - s, N_DEV)
            recv_idx = jnp.mod(my - s, N_DEV)

        send_buf[slot] = out_ref[pl.ds(send_idx * CHUNK, CHUNK), :].astype(
            jnp.bfloat16)
        if t >= 1:
            pl.semaphore_wait(credit_sem, 1)
        rdma = pltpu.make_async_remote_copy(
            src_ref=send_buf.at[slot],
            dst_ref=recv_buf.at[slot],
            send_sem=send_sems.at[slot],
            recv_sem=recv_sems.at[slot],
            device_id=(right,),
            device_id_type=pl.DeviceIdType.MESH,
        )
        rdma.start()
        rdma.wait()

        if t < N_DEV - 1:
            out_ref[pl.ds(recv_idx * CHUNK, CHUNK), :] += (
                recv_buf[slot].astype(jnp.float32))
        else:
            out_ref[pl.ds(recv_idx * CHUNK, CHUNK), :] = (
                recv_buf[slot].astype(jnp.float32))
        if t < 2 * (N_DEV - 1) - 1:
            pl.semaphore_signal(credit_sem, inc=1, device_id=(left,),
                                device_id_type=pl.DeviceIdType.MESH)


def kernel(x, Wg, Wu, Wd):
    m, k = x.shape
    h_per = Wg.shape[1]
    d = Wd.shape[1]
    n_steps = h_per // HB

    x_bf = x.astype(jnp.bfloat16)

    partial = pl.pallas_call(
        _compute_body,
        grid=(n_steps,),
        in_specs=[
            pl.BlockSpec((m, k), lambda i: (0, 0)),
            pl.BlockSpec((k, HB), lambda i: (0, i)),
            pl.BlockSpec((k, HB), lambda i: (0, i)),
            pl.BlockSpec((HB, d), lambda i: (i, 0)),
        ],
        out_specs=pl.BlockSpec((m, d), lambda i: (0, 0)),
        out_shape=jax.ShapeDtypeStruct((m, d), jnp.float32),
        compiler_params=pltpu.CompilerParams(
            dimension_semantics=("arbitrary",),
            vmem_limit_bytes=60 * 1024 * 1024),
    )(x_bf, Wg, Wu, Wd)

    return pl.pallas_call(
        _allreduce_body,
        out_shape=jax.ShapeDtypeStruct((m, d), jnp.float32),
        in_specs=[pl.BlockSpec(memory_space=pltpu.VMEM)],
        out_specs=pl.BlockSpec(memory_space=pltpu.VMEM),
        scratch_shapes=[
            pltpu.VMEM((2, CHUNK, d), jnp.bfloat16),
            pltpu.VMEM((2, CHUNK, d), jnp.bfloat16),
            pltpu.SemaphoreType.DMA((2,)),
            pltpu.SemaphoreType.DMA((2,)),
            pltpu.SemaphoreType.REGULAR,
        ],
        compiler_params=pltpu.CompilerParams(
            collective_id=0, vmem_limit_bytes=60 * 1024 * 1024),
    )(partial)


# device time: 314764 ns/iter; 1.2537x vs baseline; 1.2537x over previous
import functools

import jax
import jax.numpy as jnp
from jax import lax
from jax.experimental import pallas as pl
from jax.experimental.pallas import tpu as pltpu

N_DEV = 8
M = 2048
D = 2048
HB = 256
CHUNK = M // N_DEV


def _compute_body(x_ref, wg_ref, wu_ref, wd_ref, out_ref):
    k = pl.program_id(0)
    xb = x_ref[...]
    g = jnp.dot(xb, wg_ref[...].astype(jnp.bfloat16),
                preferred_element_type=jnp.float32)
    u = jnp.dot(xb, wu_ref[...].astype(jnp.bfloat16),
                preferred_element_type=jnp.float32)
    h = (g * (u * (1.0 / (1.0 + jnp.exp(-u))))).astype(jnp.bfloat16)
    p = jnp.dot(h, wd_ref[...].astype(jnp.bfloat16),
                preferred_element_type=jnp.float32)

    @pl.when(k == 0)
    def _():
        out_ref[...] = p

    @pl.when(k != 0)
    def _():
        out_ref[...] += p


N_HOPS = 2 * (N_DEV - 1)
HALF = D // 2


def _allreduce_body(p_ref, out_ref, send_r, recv_r, send_l, recv_l,
                    ssem_r, rsem_r, ssem_l, rsem_l, credit_r, credit_l):
    my = lax.axis_index("i")
    left = jnp.mod(my - 1, N_DEV)
    right = jnp.mod(my + 1, N_DEV)

    dirs = [
        dict(dst=right, upstream=left, sbuf=send_r, rbuf=recv_r,
             ssem=ssem_r, rsem=rsem_r, credit=credit_r, c0=0,
             rs_send=lambda t: jnp.mod(my - t, N_DEV),
             rs_recv=lambda t: jnp.mod(my - t - 1, N_DEV),
             ag_send=lambda s: jnp.mod(my + 1 - s, N_DEV),
             ag_recv=lambda s: jnp.mod(my - s, N_DEV),
             descs=[None] * N_HOPS),
        dict(dst=left, upstream=right, sbuf=send_l, rbuf=recv_l,
             ssem=ssem_l, rsem=rsem_l, credit=credit_l, c0=HALF,
             rs_send=lambda t: jnp.mod(my + t, N_DEV),
             rs_recv=lambda t: jnp.mod(my + t + 1, N_DEV),
             ag_send=lambda s: jnp.mod(my - 1 + s, N_DEV),
             ag_recv=lambda s: jnp.mod(my + s, N_DEV),
             descs=[None] * N_HOPS),
    ]

    def acc_chunk(d, idx):
        return out_ref[pl.ds(idx * CHUNK, CHUNK), d["c0"]:d["c0"] + HALF]

    def set_acc_chunk(d, idx, val):
        out_ref[pl.ds(idx * CHUNK, CHUNK), d["c0"]:d["c0"] + HALF] = val

    barrier = pltpu.get_barrier_semaphore()
    for nbr in (left, right):
        pl.semaphore_signal(barrier, inc=1, device_id=(nbr,),
                            device_id_type=pl.DeviceIdType.MESH)
    pl.semaphore_wait(barrier, 2)

    out_ref[...] = p_ref[...]

    for t in range(N_HOPS):
        slot = t % 2
        for d in dirs:
            if t < N_DEV - 1:
                sidx = d["rs_send"](t)
            else:
                sidx = d["ag_send"](t - (N_DEV - 1))
            d["sbuf"][slot] = acc_chunk(d, sidx).astype(jnp.bfloat16)
            if t >= 1:
                pl.semaphore_wait(d["credit"], 1)
            rdma = pltpu.make_async_remote_copy(
                src_ref=d["sbuf"].at[slot],
                dst_ref=d["rbuf"].at[slot],
                send_sem=d["ssem"].at[slot],
                recv_sem=d["rsem"].at[slot],
                device_id=(d["dst"],),
                device_id_type=pl.DeviceIdType.MESH,
            )
            rdma.start()
            d["descs"][t] = rdma
        for d in dirs:
            d["descs"][t].wait()
            if t < N_DEV - 1:
                ridx = d["rs_recv"](t)
                set_acc_chunk(d, ridx,
                              acc_chunk(d, ridx)
                              + d["rbuf"][slot].astype(jnp.float32))
            else:
                set_acc_chunk(d, d["ag_recv"](t - (N_DEV - 1)),
                              d["rbuf"][slot].astype(jnp.float32))
            if t < N_HOPS - 1:
                pl.semaphore_signal(d["credit"], inc=1,
                                    device_id=(d["upstream"],),
                                    device_id_type=pl.DeviceIdType.MESH)


def kernel(x, Wg, Wu, Wd):
    m, k = x.shape
    h_per = Wg.shape[1]
    d = Wd.shape[1]
    n_steps = h_per // HB

    x_bf = x.astype(jnp.bfloat16)

    partial = pl.pallas_call(
        _compute_body,
        grid=(n_steps,),
        in_specs=[
            pl.BlockSpec((m, k), lambda i: (0, 0)),
            pl.BlockSpec((k, HB), lambda i: (0, i)),
            pl.BlockSpec((k, HB), lambda i: (0, i)),
            pl.BlockSpec((HB, d), lambda i: (i, 0)),
        ],
        out_specs=pl.BlockSpec((m, d), lambda i: (0, 0)),
        out_shape=jax.ShapeDtypeStruct((m, d), jnp.float32),
        compiler_params=pltpu.CompilerParams(
            dimension_semantics=("arbitrary",),
            vmem_limit_bytes=60 * 1024 * 1024),
    )(x_bf, Wg, Wu, Wd)

    return pl.pallas_call(
        _allreduce_body,
        out_shape=jax.ShapeDtypeStruct((m, d), jnp.float32),
        in_specs=[pl.BlockSpec(memory_space=pltpu.VMEM)],
        out_specs=pl.BlockSpec(memory_space=pltpu.VMEM),
        scratch_shapes=[
            pltpu.VMEM((2, CHUNK, HALF), jnp.bfloat16),
            pltpu.VMEM((2, CHUNK, HALF), jnp.bfloat16),
            pltpu.VMEM((2, CHUNK, HALF), jnp.bfloat16),
            pltpu.VMEM((2, CHUNK, HALF), jnp.bfloat16),
            pltpu.SemaphoreType.DMA((2,)),
            pltpu.SemaphoreType.DMA((2,)),
            pltpu.SemaphoreType.DMA((2,)),
            pltpu.SemaphoreType.DMA((2,)),
            pltpu.SemaphoreType.REGULAR,
            pltpu.SemaphoreType.REGULAR,
        ],
        compiler_params=pltpu.CompilerParams(
            collective_id=0, vmem_limit_bytes=60 * 1024 * 1024),
    )(partial)


# device time: 259566 ns/iter; 1.5204x vs baseline; 1.2127x over previous
import functools

import jax
import jax.numpy as jnp
from jax import lax
from jax.experimental import pallas as pl
from jax.experimental.pallas import tpu as pltpu

N_DEV = 8
M = 2048
D = 2048
HB = 256
CHUNK = M // N_DEV


def _compute_body(x_ref, wg_ref, wu_ref, wd_ref, out_ref):
    k = pl.program_id(0)
    xb = x_ref[...]
    g = jnp.dot(xb, wg_ref[...].astype(jnp.bfloat16),
                preferred_element_type=jnp.float32)
    u = jnp.dot(xb, wu_ref[...].astype(jnp.bfloat16),
                preferred_element_type=jnp.float32)
    h = (g * (u * (1.0 / (1.0 + jnp.exp(-u))))).astype(jnp.bfloat16)
    p = jnp.dot(h, wd_ref[...].astype(jnp.bfloat16),
                preferred_element_type=jnp.float32)

    @pl.when(k == 0)
    def _():
        out_ref[...] = p

    @pl.when(k != 0)
    def _():
        out_ref[...] += p


_ROUND_ROWS = (1024, 512, 256, 256, 512, 1024)
_ROFF = (0, 1024, 1536, 1792, 2048, 2560)
_RBUF_ROWS = 3584
_PARTS = (
    (0, 768, ("x", "y", "z")),
    (768, 640, ("y", "z", "x")),
    (1408, 640, ("z", "x", "y")),
)


def _allreduce_body(p_ref, out_ref, sbuf0, rbuf0, sbuf1, rbuf1, sbuf2, rbuf2,
                    ssem0, rsem0, ssem1, rsem1, ssem2, rsem2, copy_sem):
    my = lax.axis_index("i")
    z = my // 4
    p = my % 4
    yb = p // 2
    xb = jnp.bitwise_xor(p % 2, yb)
    nx = 1 - xb
    x_partner = z * 4 + (yb * 2 + jnp.bitwise_xor(nx, yb))
    ny = 1 - yb
    y_partner = z * 4 + (ny * 2 + jnp.bitwise_xor(xb, ny))
    z_partner = my + (1 - 2 * z) * 4
    ax = {"x": (x_partner, xb), "y": (y_partner, yb), "z": (z_partner, z)}

    parts = []
    for (c0, cw, order), sbuf, rbuf, ssem, rsem in zip(
            _PARTS, (sbuf0, sbuf1, sbuf2), (rbuf0, rbuf1, rbuf2),
            (ssem0, ssem1, ssem2), (rsem0, rsem1, rsem2)):
        parts.append(dict(c0=c0, cw=cw, order=order, sbuf=sbuf, rbuf=rbuf,
                          ssem=ssem, rsem=rsem, off=jnp.int32(0),
                          descs=[None] * 6))

    cp = pltpu.make_async_copy(p_ref, out_ref, copy_sem)
    cp.start()

    barrier = pltpu.get_barrier_semaphore()
    for nbr, _ in ax.values():
        pl.semaphore_signal(barrier, inc=1, device_id=(nbr,),
                            device_id_type=pl.DeviceIdType.MESH)
    pl.semaphore_wait(barrier, 3)
    cp.wait()

    for r in range(6):
        s = _ROUND_ROWS[r]
        for pt in parts:
            axis = pt["order"][r] if r < 3 else pt["order"][5 - r]
            partner, bit = ax[axis]
            if r < 3:
                send_off = pt["off"] + (1 - bit) * s
                pt["off"] = pt["off"] + bit * s
            else:
                send_off = pt["off"]
            slot = r % 2
            if r >= 2:
                pt["descs"][r - 2].wait_send()
            pt["sbuf"][slot, :s, :] = out_ref[
                pl.ds(send_off, s), pt["c0"]:pt["c0"] + pt["cw"]].astype(
                    jnp.bfloat16)
            rdma = pltpu.make_async_remote_copy(
                src_ref=pt["sbuf"].at[slot, pl.ds(0, s), :],
                dst_ref=pt["rbuf"].at[pl.ds(_ROFF[r], s), :],
                send_sem=pt["ssem"].at[r],
                recv_sem=pt["rsem"].at[r],
                device_id=(partner,),
                device_id_type=pl.DeviceIdType.MESH,
            )
            rdma.start()
            pt["descs"][r] = rdma
            pt["_bit"] = bit
        for pt in parts:
            pt["descs"][r].wait_recv()
            recv = pt["rbuf"][pl.ds(_ROFF[r], s), :].astype(jnp.float32)
            cols = slice(pt["c0"], pt["c0"] + pt["cw"])
            if r < 3:
                out_ref[pl.ds(pt["off"], s), cols] += recv
            else:
                off_p = pt["off"] + (1 - 2 * pt["_bit"]) * s
                out_ref[pl.ds(off_p, s), cols] = recv
                pt["off"] = pt["off"] - pt["_bit"] * s

    for pt in parts:
        pt["descs"][4].wait_send()
        pt["descs"][5].wait_send()


def kernel(x, Wg, Wu, Wd):
    m, k = x.shape
    h_per = Wg.shape[1]
    d = Wd.shape[1]
    n_steps = h_per // HB

    x_bf = x.astype(jnp.bfloat16)

    partial = pl.pallas_call(
        _compute_body,
        grid=(n_steps,),
        in_specs=[
            pl.BlockSpec((m, k), lambda i: (0, 0)),
            pl.BlockSpec((k, HB), lambda i: (0, i)),
            pl.BlockSpec((k, HB), lambda i: (0, i)),
            pl.BlockSpec((HB, d), lambda i: (i, 0)),
        ],
        out_specs=pl.BlockSpec((m, d), lambda i: (0, 0)),
        out_shape=jax.ShapeDtypeStruct((m, d), jnp.float32),
        compiler_params=pltpu.CompilerParams(
            dimension_semantics=("arbitrary",),
            vmem_limit_bytes=60 * 1024 * 1024),
    )(x_bf, Wg, Wu, Wd)

    scratch = []
    for _, cw, _ in _PARTS:
        scratch.append(pltpu.VMEM((2, 1024, cw), jnp.bfloat16))
        scratch.append(pltpu.VMEM((_RBUF_ROWS, cw), jnp.bfloat16))
    for _ in _PARTS:
        scratch.append(pltpu.SemaphoreType.DMA((6,)))
        scratch.append(pltpu.SemaphoreType.DMA((6,)))
    scratch.append(pltpu.SemaphoreType.DMA)

    return pl.pallas_call(
        _allreduce_body,
        out_shape=jax.ShapeDtypeStruct((m, d), jnp.float32),
        in_specs=[pl.BlockSpec(memory_space=pl.ANY)],
        out_specs=pl.BlockSpec(memory_space=pltpu.VMEM),
        scratch_shapes=scratch,
        compiler_params=pltpu.CompilerParams(
            collective_id=0, vmem_limit_bytes=60 * 1024 * 1024),
    )(partial)


# device time: 221125 ns/iter; 1.7847x vs baseline; 1.1738x over previous
import jax
import jax.numpy as jnp
from jax import lax
from jax.experimental import pallas as pl
from jax.experimental.pallas import tpu as pltpu

N_DEV = 8
M = 2048
D = 2048
HB = 512
WAVES = 4
RHALF = M // WAVES

_RS_S = (RHALF // 2, RHALF // 4, RHALF // 8)
_AG_S = (RHALF // 8, RHALF // 4, RHALF // 2)
_ROFF = (0, RHALF // 2, 3 * RHALF // 4, 7 * RHALF // 8,
         RHALF, 5 * RHALF // 4)
_RBUF_ROWS = 7 * RHALF // 4
_PARTS = (
    (0, 768, ("x", "y", "z")),
    (768, 640, ("y", "z", "x")),
    (1408, 640, ("z", "x", "y")),
)


def _axis_info(my):
    z = my // 4
    p = my % 4
    yb = p // 2
    xb = jnp.bitwise_xor(p % 2, yb)
    nx = 1 - xb
    x_partner = z * 4 + (yb * 2 + jnp.bitwise_xor(nx, yb))
    ny = 1 - yb
    y_partner = z * 4 + (ny * 2 + jnp.bitwise_xor(xb, ny))
    z_partner = my + (1 - 2 * z) * 4
    return {"x": (x_partner, xb), "y": (y_partner, yb), "z": (z_partner, z)}


def _round_geom(pt, ax, hf, j):
    base = hf * RHALF
    order = pt["order"]
    if j < 3:
        axis = order[j]
        s = _RS_S[j]
        off_b = base
        for i in range(j):
            off_b = off_b + ax[order[i]][1] * _RS_S[i]
        partner, bit = ax[axis]
        send_off = off_b + (1 - bit) * s
        cons_off = off_b + bit * s
        is_rs = True
    else:
        jj = j - 3
        axis = order[2 - jj]
        s = _AG_S[jj]
        off_b = base
        for i in range(3 - jj):
            off_b = off_b + ax[order[i]][1] * _RS_S[i]
        partner, bit = ax[axis]
        send_off = off_b
        cons_off = off_b + (1 - 2 * bit) * s
        is_rs = False
    return dict(s=s, partner=partner, send_off=send_off, cons_off=cons_off,
                is_rs=is_rs)


def _send_desc(pt, hf, j, geom):
    k = hf * 6 + j
    return pltpu.make_async_remote_copy(
        src_ref=pt["sbuf"].at[k % 2, pl.ds(0, geom["s"]), :],
        dst_ref=pt["rbuf"].at[pl.ds(_ROFF[j], geom["s"]), :],
        send_sem=pt["ssem"].at[k],
        recv_sem=pt["rsem"].at[k],
        device_id=(geom["partner"],),
        device_id_type=pl.DeviceIdType.MESH,
    )


def _stage_round(pt, ax, out_ref, hf, j):
    geom = _round_geom(pt, ax, hf, j)
    k = hf * 6 + j
    if k >= 2:
        pg = _round_geom(pt, ax, (k - 2) // 6, (k - 2) % 6)
        _send_desc(pt, (k - 2) // 6, (k - 2) % 6, pg).wait_send()
    cols = slice(pt["c0"], pt["c0"] + pt["cw"])
    pt["sbuf"][k % 2, :geom["s"], :] = out_ref[
        pl.ds(geom["send_off"], geom["s"]), cols].astype(jnp.bfloat16)
    _send_desc(pt, hf, j, geom).start()


def _consume_round(pt, ax, out_ref, hf, j):
    geom = _round_geom(pt, ax, hf, j)
    _send_desc(pt, hf, j, geom).wait_recv()
    cols = slice(pt["c0"], pt["c0"] + pt["cw"])
    recv = pt["rbuf"][pl.ds(_ROFF[j], geom["s"]), :].astype(jnp.float32)
    if geom["is_rs"]:
        out_ref[pl.ds(geom["cons_off"], geom["s"]), cols] += recv
    else:
        out_ref[pl.ds(geom["cons_off"], geom["s"]), cols] = recv


def _fused_body(x_ref, wg_ref, wu_ref, wd_ref, out_ref,
                sbuf0, rbuf0, sbuf1, rbuf1, sbuf2, rbuf2,
                ssem0, rsem0, ssem1, rsem1, ssem2, rsem2):
    r = pl.program_id(0)
    h = pl.program_id(1)
    n_h = pl.num_programs(1)
    my = lax.axis_index("i")
    ax = _axis_info(my)

    parts = []
    for (c0, cw, order), sbuf, rbuf, ssem, rsem in zip(
            _PARTS, (sbuf0, sbuf1, sbuf2), (rbuf0, rbuf1, rbuf2),
            (ssem0, ssem1, ssem2), (rsem0, rsem1, rsem2)):
        parts.append(dict(c0=c0, cw=cw, order=order, sbuf=sbuf, rbuf=rbuf,
                          ssem=ssem, rsem=rsem))

    @pl.when(jnp.logical_and(r == 0, h == 0))
    def _():
        barrier = pltpu.get_barrier_semaphore()
        for nbr, _ in ax.values():
            pl.semaphore_signal(barrier, inc=1, device_id=(nbr,),
                                device_id_type=pl.DeviceIdType.MESH)
        pl.semaphore_wait(barrier, 3)

    xb = x_ref[...]
    g = jnp.dot(xb, wg_ref[...].astype(jnp.bfloat16),
                preferred_element_type=jnp.float32)
    u = jnp.dot(xb, wu_ref[...].astype(jnp.bfloat16),
                preferred_element_type=jnp.float32)
    hh = (g * (u * (1.0 / (1.0 + jnp.exp(-u))))).astype(jnp.bfloat16)
    p = jnp.dot(hh, wd_ref[...].astype(jnp.bfloat16),
                preferred_element_type=jnp.float32)

    @pl.when(h == 0)
    def _():
        out_ref[pl.ds(r * RHALF, RHALF), :] = p

    @pl.when(h != 0)
    def _():
        out_ref[pl.ds(r * RHALF, RHALF), :] += p

    for w in range(WAVES - 1):
        for j in range(6):
            @pl.when(jnp.logical_and(r == w + 1, h == j))
            def _(w=w, j=j):
                for pt in parts:
                    if j >= 1:
                        _consume_round(pt, ax, out_ref, w, j - 1)
                for pt in parts:
                    _stage_round(pt, ax, out_ref, w, j)

        @pl.when(jnp.logical_and(r == w + 1, h == 6))
        def _(w=w):
            for pt in parts:
                _consume_round(pt, ax, out_ref, w, 5)

    @pl.when(jnp.logical_and(r == WAVES - 1, h == n_h - 1))
    def _():
        for j in range(6):
            for pt in parts:
                _stage_round(pt, ax, out_ref, WAVES - 1, j)
            for pt in parts:
                _consume_round(pt, ax, out_ref, WAVES - 1, j)
        for pt in parts:
            for k in (6 * WAVES - 2, 6 * WAVES - 1):
                pg = _round_geom(pt, ax, k // 6, k % 6)
                _send_desc(pt, k // 6, k % 6, pg).wait_send()


def kernel(x, Wg, Wu, Wd):
    m, kdim = x.shape
    h_per = Wg.shape[1]
    d = Wd.shape[1]
    n_h = h_per // HB

    x_bf = x.astype(jnp.bfloat16)

    scratch = []
    for _, cw, _ in _PARTS:
        scratch.append(pltpu.VMEM((2, RHALF // 2, cw), jnp.bfloat16))
        scratch.append(pltpu.VMEM((_RBUF_ROWS, cw), jnp.bfloat16))
    for _ in _PARTS:
        scratch.append(pltpu.SemaphoreType.DMA((6 * WAVES,)))
        scratch.append(pltpu.SemaphoreType.DMA((6 * WAVES,)))

    return pl.pallas_call(
        _fused_body,
        grid=(WAVES, n_h),
        in_specs=[
            pl.BlockSpec((RHALF, kdim), lambda r, i: (r, 0)),
            pl.BlockSpec((kdim, HB), lambda r, i: (0, i)),
            pl.BlockSpec((kdim, HB), lambda r, i: (0, i)),
            pl.BlockSpec((HB, d), lambda r, i: (i, 0)),
        ],
        out_specs=pl.BlockSpec((m, d), lambda r, i: (0, 0)),
        out_shape=jax.ShapeDtypeStruct((m, d), jnp.float32),
        scratch_shapes=scratch,
        compiler_params=pltpu.CompilerParams(
            dimension_semantics=("arbitrary", "arbitrary"),
            collective_id=0,
            vmem_limit_bytes=64 * 1024 * 1024),
    )(x_bf, Wg, Wu, Wd)
